# Initial kernel scaffold; baseline (speedup 1.0000x reference)
import jax
import jax.numpy as jnp
from jax.experimental import pallas as pl


def kernel(x, edge_index, Wl1, bl1, Wr1, Wl2, bl2, Wr2, W_lin1, b_lin1, W_last, b_last):
    raise NotImplementedError("write your pallas kernel here")


# speedup vs baseline: 5.5949x; 5.5949x over previous
import functools

import jax
import jax.numpy as jnp
import numpy as np
from jax import lax
from jax.experimental import pallas as pl
from jax.experimental.pallas import tpu as pltpu
from jax.experimental.pallas import tpu_sc as plsc

N = 100000
E = 1600000
H = 128
NG = 8
GW = 16
LANES = 128
ROWS = 12544
EP = ROWS * LANES
ACC_ROWS = 102400
NP = 100096
NSUB = 16
ZCH = ACC_ROWS // NSUB // 128
OUT_PT = NP // NSUB
C1_ROWS = ROWS // 2
C1_PT = C1_ROWS // NSUB
C2_PT = ROWS // NSUB
CH = 8

_mesh = functools.partial(
    plsc.VectorSubcoreMesh, core_axis_name="c", subcore_axis_name="s",
    num_cores=2, num_subcores=NSUB)


def _zero_zbuf(zbuf):
    z = jnp.zeros((GW,), jnp.float32)
    for i in range(128):
        zbuf[i] = z


def _zero_accum(accum, zbuf, s):
    for k in range(ZCH):
        pltpu.sync_copy(zbuf, accum.at[pl.ds(s * (ZCH * 128) + k * 128, 128)])


def _edge_chunk(tab, srcr, dstr, accum, src_v, dst_v, rows_v, sem, r0):
    pltpu.sync_copy(srcr.at[pl.ds(r0, CH)], src_v)
    pltpu.sync_copy(dstr.at[pl.ds(r0, CH)], dst_v)
    cps = [
        pltpu.async_copy(tab.at[src_v.at[j]],
                         rows_v.at[pl.ds(j * LANES, LANES)], sem)
        for j in range(CH)
    ]
    for cp in cps:
        cp.wait()
    for j in range(CH):
        pltpu.sync_copy(rows_v.at[pl.ds(j * LANES, LANES)],
                        accum.at[dst_v.at[j]], add=True)


def _conv1_body(xaug, srcr, dstr, out, accum, src_v, dst_v, rows_v, zbuf, sem):
    c = lax.axis_index("c")
    s = lax.axis_index("s")
    _zero_zbuf(zbuf)
    _zero_accum(accum, zbuf, s)
    plsc.subcore_barrier()

    base = c * C1_ROWS + s * C1_PT

    def chunk(i, carry):
        _edge_chunk(xaug, srcr, dstr, accum, src_v, dst_v, rows_v, sem,
                    base + i * CH)
        return carry

    lax.fori_loop(0, C1_PT // CH, chunk, 0)
    plsc.subcore_barrier()
    pltpu.sync_copy(accum.at[pl.ds(s * OUT_PT, OUT_PT)],
                    out.at[pl.ds(c * NP + s * OUT_PT, OUT_PT)])


def _conv2_body(t0, t1, t2, t3, t4, t5, t6, t7, srcr, dstr, out,
                accum, src_v, dst_v, rows_v, zbuf, sem):
    c = lax.axis_index("c")
    s = lax.axis_index("s")
    tabs = [t0, t1, t2, t3, t4, t5, t6, t7]
    _zero_zbuf(zbuf)
    for g in range(NG):

        @pl.when(c == g // 4)
        def _():
            _zero_accum(accum, zbuf, s)

        plsc.subcore_barrier()

        @pl.when(c == g // 4)
        def _():
            base = s * C2_PT

            def chunk(i, carry):
                _edge_chunk(tabs[g], srcr, dstr, accum, src_v, dst_v,
                            rows_v, sem, base + i * CH)
                return carry

            lax.fori_loop(0, C2_PT // CH, chunk, 0)

        plsc.subcore_barrier()

        @pl.when(c == g // 4)
        def _():
            pltpu.sync_copy(accum.at[pl.ds(s * OUT_PT, OUT_PT)],
                            out.at[pl.ds(g * NP + s * OUT_PT, OUT_PT)])

        plsc.subcore_barrier()


def _sc_scratch():
    return [
        pltpu.VMEM_SHARED((ACC_ROWS, GW), jnp.float32),
        pltpu.VMEM((CH, LANES), jnp.int32),
        pltpu.VMEM((CH, LANES), jnp.int32),
        pltpu.VMEM((CH * LANES, GW), jnp.float32),
        pltpu.VMEM((128, GW), jnp.float32),
        pltpu.SemaphoreType.DMA,
    ]


_conv1_agg = pl.kernel(
    _conv1_body,
    out_type=jax.ShapeDtypeStruct((2 * NP, GW), jnp.float32),
    mesh=_mesh(),
    scratch_types=_sc_scratch(),
    compiler_params=pltpu.CompilerParams(use_tc_tiling_on_sc=False),
)

_conv2_agg = pl.kernel(
    _conv2_body,
    out_type=jax.ShapeDtypeStruct((NG * NP, GW), jnp.float32),
    mesh=_mesh(),
    scratch_types=_sc_scratch(),
    compiler_params=pltpu.CompilerParams(use_tc_tiling_on_sc=False),
)


BLK = 1472


def _stage_b_body(p0, p1, xaug, wl, wr, bl, *outs):
    psum = p0[...] + p1[...]
    cnt = psum[:, 3:4]
    rcnt = 1.0 / jnp.maximum(cnt, 1.0)
    mean16 = psum * rcnt
    z = (jnp.dot(mean16, wl[...], preferred_element_type=jnp.float32)
         + jnp.dot(xaug[...], wr[...], preferred_element_type=jnp.float32)
         + bl[...])
    h = jnp.tanh(z)
    for g in range(NG):
        outs[g][...] = h[:, g * GW:(g + 1) * GW]
    outs[NG][...] = jnp.broadcast_to(rcnt, (BLK, GW))


def _stage_c_body(s2, rc, wl2, wr2, bl2, wlin, blin, wlast, blast, *rest):
    tabs = rest[:NG]
    out = rest[NG]
    rcv = rc[...]
    mean2 = jnp.concatenate([s2[g] * rcv for g in range(NG)], axis=1)
    h1 = jnp.concatenate([t[...] for t in tabs], axis=1)
    z2 = (jnp.dot(mean2, wl2[...], preferred_element_type=jnp.float32)
          + jnp.dot(h1, wr2[...], preferred_element_type=jnp.float32)
          + bl2[...])
    h2 = jnp.tanh(z2)
    z3 = (jnp.dot(h2, wlin[...], preferred_element_type=jnp.float32)
          + blin[...])
    h3 = jnp.tanh(z3)
    z4 = (jnp.dot(h3, wlast[...], preferred_element_type=jnp.float32)
          + blast[...])
    m = jnp.max(z4, axis=1, keepdims=True)
    e = jnp.exp(z4 - m)
    out[...] = e / jnp.sum(e, axis=1, keepdims=True)


def _blk(i):
    return (i, 0)


def _rep(i):
    return (0, 0)


def _normalize_x(x):
    coords = x[:, :2]
    areas = x[:, -1:]
    max_c = jnp.max(coords, axis=0)
    min_c = jnp.min(coords, axis=0)
    theta = jnp.float32(np.pi / 2)
    R = jnp.array([[jnp.cos(theta), -jnp.sin(theta)],
                   [jnp.sin(theta), jnp.cos(theta)]], dtype=jnp.float32)
    rotated = (R @ coords.T).T
    cond = (max_c[1] - min_c[1]) > (max_c[0] - min_c[0])
    coords = jnp.where(cond, rotated, coords)
    coords = (coords - jnp.mean(coords, axis=0)) / jnp.max(coords, axis=0)
    areas = areas / jnp.max(areas, axis=0)
    return jnp.concatenate([coords, areas], axis=-1)


def kernel(x, edge_index, Wl1, bl1, Wr1, Wl2, bl2, Wr2, W_lin1, b_lin1,
           W_last, b_last):
    xn = _normalize_x(x)
    xaug = jnp.concatenate(
        [xn, jnp.ones((N, 1), jnp.float32), jnp.zeros((N, GW - 4), jnp.float32)],
        axis=1)
    xaug_p = jnp.pad(xaug, ((0, NP - N), (0, 0)))

    pad = EP - E
    srcp = jnp.concatenate(
        [edge_index[0], jnp.zeros((pad,), jnp.int32)]).reshape(ROWS, LANES)
    dstp = jnp.concatenate(
        [edge_index[1], jnp.full((pad,), N, jnp.int32)]).reshape(ROWS, LANES)

    Wl1p = jnp.zeros((GW, H), jnp.float32).at[:3].set(Wl1)
    Wr1p = jnp.zeros((GW, H), jnp.float32).at[:3].set(Wr1)

    p = _conv1_agg(xaug, srcp, dstp).reshape(2, NP, GW)

    nblk = NP // BLK
    grid = (nblk,)
    sds16 = jax.ShapeDtypeStruct((NP, GW), jnp.float32)
    bs16 = pl.BlockSpec((BLK, GW), _blk)

    stage_b = pl.pallas_call(
        _stage_b_body,
        grid=grid,
        in_specs=[bs16, bs16, bs16,
                  pl.BlockSpec((GW, H), _rep),
                  pl.BlockSpec((GW, H), _rep),
                  pl.BlockSpec((1, H), _rep)],
        out_specs=[bs16] * (NG + 1),
        out_shape=[sds16] * (NG + 1),
    )
    *tabs, rc = stage_b(p[0], p[1], xaug_p, Wl1p, Wr1p, bl1.reshape(1, H))

    s2 = _conv2_agg(*tabs, srcp, dstp).reshape(NG, NP, GW)

    cdim = W_last.shape[1]
    stage_c = pl.pallas_call(
        _stage_c_body,
        grid=grid,
        in_specs=[pl.BlockSpec((NG, BLK, GW), lambda i: (0, i, 0)),
                  bs16,
                  pl.BlockSpec((H, H), _rep),
                  pl.BlockSpec((H, H), _rep),
                  pl.BlockSpec((1, H), _rep),
                  pl.BlockSpec((H, H), _rep),
                  pl.BlockSpec((1, H), _rep),
                  pl.BlockSpec((H, cdim), _rep),
                  pl.BlockSpec((1, cdim), _rep)]
                 + [bs16] * NG,
        out_specs=pl.BlockSpec((BLK, cdim), _blk),
        out_shape=jax.ShapeDtypeStruct((NP, cdim), jnp.float32),
    )
    full = stage_c(s2, rc, Wl2, Wr2, bl2.reshape(1, H),
                   W_lin1, b_lin1.reshape(1, H),
                   W_last, b_last.reshape(1, cdim), *tabs)
    return full[:N]
